# Initial kernel scaffold; baseline (speedup 1.0000x reference)
import jax
import jax.numpy as jnp
from jax.experimental import pallas as pl


def kernel(input_ids, word_embeddings, position_embeddings):
    raise NotImplementedError("write your pallas kernel here")


# speedup vs baseline: 1.6029x; 1.6029x over previous
import functools

import jax
import jax.numpy as jnp
from jax import lax
from jax.experimental import pallas as pl
from jax.experimental.pallas import tpu as pltpu
from jax.experimental.pallas import tpu_sc as plsc

_VOCAB = 50265
_HIDDEN = 768
_MAX_POS = 512
_BATCH = 128
_SEQ = 512

_NC = 2
_NS = 16
_NW = _NC * _NS
_SEQ_PER_W = _BATCH // _NW
_C = 64
_NCHUNK = _SEQ // _C
_LANES = 16


def _emb_body(ids_hbm, words_hbm, pos_hbm, out_hbm, idx_v, w_v, p_v, sem):
    wid = lax.axis_index("s") * _NC + lax.axis_index("c")

    def chunk_body(cj, _):
        pltpu.sync_copy(pos_hbm.at[pl.ds(cj * _C, _C)], p_v)

        def row_body(r, _):
            base = (wid * _SEQ_PER_W + r) * _SEQ + cj * _C
            pltpu.sync_copy(ids_hbm.at[pl.ds(base, _C)], idx_v)
            pltpu.async_copy(words_hbm.at[idx_v], w_v, sem).wait()

            def add_body(i, _):
                for j in range(_HIDDEN // _LANES):
                    sl = pl.ds(j * _LANES, _LANES)
                    w_v[i, sl] = w_v[i, sl] + p_v[i, sl]
                return 0

            lax.fori_loop(0, _C, add_body, 0, unroll=False)
            pltpu.sync_copy(w_v, out_hbm.at[pl.ds(base, _C)])
            return 0

        lax.fori_loop(0, _SEQ_PER_W, row_body, 0, unroll=False)
        return 0

    lax.fori_loop(0, _NCHUNK, chunk_body, 0, unroll=False)


_emb_kernel = functools.partial(
    pl.kernel,
    out_type=jax.ShapeDtypeStruct((_BATCH * _SEQ, _HIDDEN), jnp.float32),
    mesh=plsc.VectorSubcoreMesh(core_axis_name="c", subcore_axis_name="s"),
    scratch_types=[
        pltpu.VMEM((_C,), jnp.int32),
        pltpu.VMEM((_C, _HIDDEN), jnp.float32),
        pltpu.VMEM((_C, _HIDDEN), jnp.float32),
        pltpu.SemaphoreType.DMA,
    ],
)(_emb_body)


def kernel(input_ids, word_embeddings, position_embeddings):
    ids_flat = input_ids.reshape(-1)
    out = _emb_kernel(ids_flat, word_embeddings, position_embeddings)
    return out.reshape(_BATCH, _SEQ, _HIDDEN)


# speedup vs baseline: 2.5722x; 1.6047x over previous
import functools

import jax
import jax.numpy as jnp
from jax import lax
from jax.experimental import pallas as pl
from jax.experimental.pallas import tpu as pltpu
from jax.experimental.pallas import tpu_sc as plsc

_VOCAB = 50265
_HIDDEN = 768
_MAX_POS = 512
_BATCH = 128
_SEQ = 512

_NC = 2
_NS = 16
_NW = _NC * _NS
_ROWS_PER_W = _BATCH // _NW
_C = 32
_NCHUNK = _SEQ // _C
_LANES = 16


def _emb_body(ids_hbm, words_hbm, pos_hbm, out_hbm,
              idx_all, p_v,
              w0, w1, w2, w3,
              sg0, sg1, sg2, sg3, so0, so1, so2, so3):
    wid = lax.axis_index("s") * _NC + lax.axis_index("c")
    w_bufs = (w0, w1, w2, w3)
    g_sems = (sg0, sg1, sg2, sg3)
    o_sems = (so0, so1, so2, so3)

    pltpu.sync_copy(ids_hbm.at[pl.ds(wid * _ROWS_PER_W * _SEQ,
                                     _ROWS_PER_W * _SEQ)], idx_all)

    def _gather(cj, b):
        idx = idx_all.at[pl.ds(b * _SEQ + cj * _C, _C)]
        return pltpu.make_async_copy(words_hbm.at[idx], w_bufs[b], g_sems[b])

    def _outwrite(cj, b):
        base = (wid * _ROWS_PER_W + b) * _SEQ + cj * _C
        return pltpu.make_async_copy(w_bufs[b], out_hbm.at[pl.ds(base, _C)],
                                     o_sems[b])

    for b in range(_ROWS_PER_W):
        _gather(0, b).start()

    def chunk_body(cj, _):
        pltpu.sync_copy(pos_hbm.at[pl.ds(cj * _C, _C)], p_v)
        for b in range(_ROWS_PER_W):
            _gather(cj, b).wait()

            def add_body(i, _):
                for j in range(_HIDDEN // _LANES):
                    sl = pl.ds(j * _LANES, _LANES)
                    w_bufs[b][i, sl] = w_bufs[b][i, sl] + p_v[i, sl]
                return 0

            lax.fori_loop(0, _C, add_body, 0, unroll=False)
            _outwrite(cj, b).start()

            @pl.when(cj > 0)
            def _():
                _outwrite(cj - 1, b).wait()

            @pl.when(cj < _NCHUNK - 1)
            def _():
                _gather(cj + 1, b).start()

        return 0

    lax.fori_loop(0, _NCHUNK, chunk_body, 0, unroll=False)
    for b in range(_ROWS_PER_W):
        _outwrite(_NCHUNK - 1, b).wait()


_emb_kernel = functools.partial(
    pl.kernel,
    out_type=jax.ShapeDtypeStruct((_BATCH * _SEQ, _HIDDEN), jnp.float32),
    mesh=plsc.VectorSubcoreMesh(core_axis_name="c", subcore_axis_name="s"),
    scratch_types=[
        pltpu.VMEM((_ROWS_PER_W * _SEQ,), jnp.int32),
        pltpu.VMEM((_C, _HIDDEN), jnp.float32),
        pltpu.VMEM((_C, _HIDDEN), jnp.float32),
        pltpu.VMEM((_C, _HIDDEN), jnp.float32),
        pltpu.VMEM((_C, _HIDDEN), jnp.float32),
        pltpu.VMEM((_C, _HIDDEN), jnp.float32),
        pltpu.SemaphoreType.DMA,
        pltpu.SemaphoreType.DMA,
        pltpu.SemaphoreType.DMA,
        pltpu.SemaphoreType.DMA,
        pltpu.SemaphoreType.DMA,
        pltpu.SemaphoreType.DMA,
        pltpu.SemaphoreType.DMA,
        pltpu.SemaphoreType.DMA,
    ],
)(_emb_body)


def kernel(input_ids, word_embeddings, position_embeddings):
    ids_flat = input_ids.reshape(-1)
    out = _emb_kernel(ids_flat, word_embeddings, position_embeddings)
    return out.reshape(_BATCH, _SEQ, _HIDDEN)
